# Initial kernel scaffold; baseline (speedup 1.0000x reference)
import jax
import jax.numpy as jnp
from jax.experimental import pallas as pl


def kernel(perm):
    raise NotImplementedError("write your pallas kernel here")


# speedup vs baseline: 6.2025x; 6.2025x over previous
import jax
import jax.numpy as jnp
from jax.experimental import pallas as pl

N = 4096
BLOCK_R = 256


def _perm_block_kernel(perm_ref, out_ref):
    p = perm_ref[0, 0, :]
    cols = jax.lax.broadcasted_iota(jnp.int32, (BLOCK_R, N), 1)
    out_ref[:, :] = (cols == p[:, None]).astype(jnp.float32)


def kernel(perm):
    perm = perm.astype(jnp.int32).reshape(N // BLOCK_R, 1, BLOCK_R)
    return pl.pallas_call(
        _perm_block_kernel,
        grid=(N // BLOCK_R,),
        in_specs=[pl.BlockSpec((1, 1, BLOCK_R), lambda i: (i, 0, 0))],
        out_specs=pl.BlockSpec((BLOCK_R, N), lambda i: (i, 0)),
        out_shape=jax.ShapeDtypeStruct((N, N), jnp.float32),
    )(perm)
